# speedup vs baseline: 10.2227x; 10.2227x over previous
import functools

import jax
import jax.numpy as jnp
from jax import lax
from jax.experimental import pallas as pl
from jax.experimental.pallas import tpu as pltpu
from jax.experimental.pallas import tpu_sc as plsc

N = 10000
E = 320000
D = 128
NC = 2
NS = 16
NW = NC * NS
CHUNK = 128
E_PAD = 327680
CH_PER_TILE = E_PAD // NW // CHUNK
N_PAD = 10240
ZROWS = 64

_MESH = plsc.VectorSubcoreMesh(core_axis_name="c", subcore_axis_name="s")


def _fill_f32(ref, n, value):
    vec = jnp.full((16,), value, jnp.float32)
    for j in range(n // 16):
        ref[pl.ds(j * 16, 16)] = vec


def _deg_body(dst_hbm, degp_hbm, deg_sh, idx_v, ones_v, zbuf_v):
    c = lax.axis_index("c")
    s = lax.axis_index("s")
    _fill_f32(ones_v, CHUNK, 1.0)
    rows = N_PAD // NS
    _fill_f32(zbuf_v, rows, 0.0)
    off = pl.multiple_of(s * rows, 8)
    pltpu.sync_copy(zbuf_v, deg_sh.at[pl.ds(off, rows)])
    plsc.subcore_barrier()

    w = c * NS + s

    def body(i, carry):
        e0 = pl.multiple_of(w * (CH_PER_TILE * CHUNK) + i * CHUNK, 8)
        pltpu.sync_copy(dst_hbm.at[pl.ds(e0, CHUNK)], idx_v)
        pltpu.sync_copy(ones_v, deg_sh.at[idx_v], add=True)
        return carry

    lax.fori_loop(0, CH_PER_TILE, body, 0)
    plsc.subcore_barrier()
    pltpu.sync_copy(deg_sh.at[pl.ds(off, rows)],
                    degp_hbm.at[c, pl.ds(off, rows)])


def _scat_body(src_hbm, dst_hbm, xs_hbm, part_hbm,
               acc_sh, sidx_v, didx_v, rows_v, zbuf_v, sem):
    c = lax.axis_index("c")
    s = lax.axis_index("s")
    zvec = jnp.zeros((16,), jnp.float32)
    for r in range(ZROWS):
        for j in range(D // 16):
            zbuf_v[r, pl.ds(j * 16, 16)] = zvec
    rows = N_PAD // NS
    for k in range(rows // ZROWS):
        off = pl.multiple_of(s * rows + k * ZROWS, 8)
        pltpu.sync_copy(zbuf_v, acc_sh.at[pl.ds(off, ZROWS)])
    plsc.subcore_barrier()

    w = c * NS + s

    def body(i, carry):
        e0 = pl.multiple_of(w * (CH_PER_TILE * CHUNK) + i * CHUNK, 8)
        pltpu.sync_copy(src_hbm.at[pl.ds(e0, CHUNK)], sidx_v)
        pltpu.sync_copy(dst_hbm.at[pl.ds(e0, CHUNK)], didx_v)
        pltpu.async_copy(xs_hbm.at[sidx_v], rows_v, sem).wait()
        pltpu.sync_copy(rows_v, acc_sh.at[didx_v], add=True)
        return carry

    lax.fori_loop(0, CH_PER_TILE, body, 0)
    plsc.subcore_barrier()
    off = pl.multiple_of(s * rows, 8)
    pltpu.sync_copy(acc_sh.at[pl.ds(off, rows)],
                    part_hbm.at[c, pl.ds(off, rows)])


def _dis(d0, d1):
    deg = d0 + d1
    return jnp.where(deg > 0, lax.rsqrt(jnp.maximum(deg, 1e-12)), 0.0)


def _scale_body(x_ref, d0_ref, d1_ref, xs_ref):
    xs_ref[...] = x_ref[...] * _dis(d0_ref[...], d1_ref[...])


def _comb_body(p0_ref, p1_ref, d0_ref, d1_ref, out_ref):
    out_ref[...] = (p0_ref[...] + p1_ref[...]) * _dis(d0_ref[...], d1_ref[...])


_deg_kernel = pl.kernel(
    _deg_body,
    out_type=jax.ShapeDtypeStruct((NC, N_PAD), jnp.float32),
    mesh=_MESH,
    scratch_types=[
        pltpu.VMEM_SHARED((N_PAD,), jnp.float32),
        pltpu.VMEM((CHUNK,), jnp.int32),
        pltpu.VMEM((CHUNK,), jnp.float32),
        pltpu.VMEM((N_PAD // NS,), jnp.float32),
    ],
)

_scat_kernel = pl.kernel(
    _scat_body,
    out_type=jax.ShapeDtypeStruct((NC, N_PAD, D), jnp.float32),
    mesh=_MESH,
    scratch_types=[
        pltpu.VMEM_SHARED((N_PAD, D), jnp.float32),
        pltpu.VMEM((CHUNK,), jnp.int32),
        pltpu.VMEM((CHUNK,), jnp.int32),
        pltpu.VMEM((CHUNK, D), jnp.float32),
        pltpu.VMEM((ZROWS, D), jnp.float32),
        pltpu.SemaphoreType.DMA,
    ],
)

_scale_call = pl.pallas_call(
    _scale_body, out_shape=jax.ShapeDtypeStruct((N, D), jnp.float32))

_comb_call = pl.pallas_call(
    _comb_body, out_shape=jax.ShapeDtypeStruct((N, D), jnp.float32))


def kernel(x, edge_index):
    ei = edge_index.astype(jnp.int32)
    src = jnp.concatenate([ei[0], jnp.zeros((E_PAD - E,), jnp.int32)])
    dst = jnp.concatenate([ei[1], jnp.full((E_PAD - E,), N_PAD - 1, jnp.int32)])

    degp = _deg_kernel(dst)
    d0 = degp[0, :N][:, None]
    d1 = degp[1, :N][:, None]

    xs = _scale_call(x, d0, d1)
    part = _scat_kernel(src, dst, xs)
    return _comb_call(part[0, :N], part[1, :N], d0, d1)


# speedup vs baseline: 12.5947x; 1.2320x over previous
import functools

import jax
import jax.numpy as jnp
from jax import lax
from jax.experimental import pallas as pl
from jax.experimental.pallas import tpu as pltpu
from jax.experimental.pallas import tpu_sc as plsc

N = 10000
E = 320000
D = 128
NC = 2
NS = 16
NW = NC * NS
CHUNK = 128
E_PAD = 327680
NG = E_PAD // NW // CHUNK
EPT = NG * CHUNK
N_PAD = 10240
ZROWS = N_PAD // NS

_MESH = plsc.VectorSubcoreMesh(core_axis_name="c", subcore_axis_name="s")


def _fill_f32(ref, n, value):
    vec = jnp.full((16,), value, jnp.float32)
    for j in range(n // 16):
        ref[pl.ds(j * 16, 16)] = vec


def _copy_chunk_idx(src_ref, dst_ref, chunk):
    for g in range(CHUNK // 16):
        dst_ref[pl.ds(g * 16, 16)] = src_ref[pl.ds(chunk * CHUNK + g * 16, 16)]


def _deg_body(dst_hbm, zeros_hbm, degp_hbm, deg_sh, didx_all, didx_c, ones_v):
    c = lax.axis_index("c")
    s = lax.axis_index("s")
    _fill_f32(ones_v, CHUNK, 1.0)
    off = pl.multiple_of(s * ZROWS, 8)
    pltpu.sync_copy(zeros_hbm, deg_sh.at[pl.ds(off, ZROWS)])
    plsc.subcore_barrier()

    w = c * NS + s
    base = pl.multiple_of(w * EPT, 8)
    pltpu.sync_copy(dst_hbm.at[pl.ds(base, EPT)], didx_all)
    for i in range(NG):
        _copy_chunk_idx(didx_all, didx_c, i)
        pltpu.sync_copy(ones_v, deg_sh.at[didx_c], add=True)

    plsc.subcore_barrier()
    pltpu.sync_copy(deg_sh.at[pl.ds(off, ZROWS)],
                    degp_hbm.at[c, pl.ds(off, ZROWS)])


def _scat_body(src_hbm, dst_hbm, xs_hbm, zeros_hbm, part_hbm,
               acc_sh, sidx_all, didx_a, didx_b,
               rows_a, rows_b, sem_a, sem_b, sem_da, sem_db):
    c = lax.axis_index("c")
    s = lax.axis_index("s")
    off = pl.multiple_of(s * ZROWS, 8)
    pltpu.sync_copy(zeros_hbm, acc_sh.at[pl.ds(off, ZROWS)])
    plsc.subcore_barrier()

    w = c * NS + s
    base = pl.multiple_of(w * EPT, 8)
    pltpu.sync_copy(src_hbm.at[pl.ds(base, EPT)], sidx_all)

    def gather_args(i):
        rows_v = rows_a if i % 2 == 0 else rows_b
        sem = sem_a if i % 2 == 0 else sem_b
        return xs_hbm.at[sidx_all.at[pl.ds(i * CHUNK, CHUNK)]], rows_v, sem

    def didx_args(i):
        didx = didx_a if i % 2 == 0 else didx_b
        sem = sem_da if i % 2 == 0 else sem_db
        return dst_hbm.at[pl.ds(base + i * CHUNK, CHUNK)], didx, sem

    pltpu.async_copy(*gather_args(0))
    pltpu.async_copy(*didx_args(0))
    for i in range(NG):
        if i + 1 < NG:
            pltpu.async_copy(*gather_args(i + 1))
            pltpu.async_copy(*didx_args(i + 1))
        pltpu.make_async_copy(*gather_args(i)).wait()
        pltpu.make_async_copy(*didx_args(i)).wait()
        rows_v = rows_a if i % 2 == 0 else rows_b
        didx_v = didx_a if i % 2 == 0 else didx_b
        pltpu.sync_copy(rows_v, acc_sh.at[didx_v], add=True)

    plsc.subcore_barrier()
    pltpu.sync_copy(acc_sh.at[pl.ds(off, ZROWS)],
                    part_hbm.at[c, pl.ds(off, ZROWS)])


def _dis(d0, d1):
    deg = d0 + d1
    return jnp.where(deg > 0, lax.rsqrt(jnp.maximum(deg, 1e-12)), 0.0)


def _scale_body(x_ref, d0_ref, d1_ref, xs_ref):
    xs_ref[...] = x_ref[...] * _dis(d0_ref[...], d1_ref[...])


def _comb_body(p0_ref, p1_ref, d0_ref, d1_ref, out_ref):
    out_ref[...] = (p0_ref[...] + p1_ref[...]) * _dis(d0_ref[...], d1_ref[...])


_deg_kernel = pl.kernel(
    _deg_body,
    out_type=jax.ShapeDtypeStruct((NC, N_PAD), jnp.float32),
    mesh=_MESH,
    scratch_types=[
        pltpu.VMEM_SHARED((N_PAD,), jnp.float32),
        pltpu.VMEM((EPT,), jnp.int32),
        pltpu.VMEM((CHUNK,), jnp.int32),
        pltpu.VMEM((CHUNK,), jnp.float32),
    ],
)

_scat_kernel = pl.kernel(
    _scat_body,
    out_type=jax.ShapeDtypeStruct((NC, N_PAD, D), jnp.float32),
    mesh=_MESH,
    scratch_types=[
        pltpu.VMEM_SHARED((N_PAD, D), jnp.float32),
        pltpu.VMEM((EPT,), jnp.int32),
        pltpu.VMEM((CHUNK,), jnp.int32),
        pltpu.VMEM((CHUNK,), jnp.int32),
        pltpu.VMEM((CHUNK, D), jnp.float32),
        pltpu.VMEM((CHUNK, D), jnp.float32),
        pltpu.SemaphoreType.DMA,
        pltpu.SemaphoreType.DMA,
        pltpu.SemaphoreType.DMA,
        pltpu.SemaphoreType.DMA,
    ],
)

_scale_call = pl.pallas_call(
    _scale_body, out_shape=jax.ShapeDtypeStruct((N, D), jnp.float32))

_comb_call = pl.pallas_call(
    _comb_body, out_shape=jax.ShapeDtypeStruct((N, D), jnp.float32))


def kernel(x, edge_index):
    ei = edge_index.astype(jnp.int32)
    npad = E_PAD - E
    src = jnp.concatenate([ei[0], jnp.zeros((npad,), jnp.int32)])
    dst = jnp.concatenate(
        [ei[1], N + jnp.arange(npad, dtype=jnp.int32) % (N_PAD - N)])
    zeros = jnp.zeros((ZROWS, D), jnp.float32)

    degp = _deg_kernel(dst, zeros[:, 0])
    d0 = degp[0, :N][:, None]
    d1 = degp[1, :N][:, None]

    xs = _scale_call(x, d0, d1)
    part = _scat_kernel(src, dst, xs, zeros)
    return _comb_call(part[0, :N], part[1, :N], d0, d1)


# speedup vs baseline: 13.5780x; 1.0781x over previous
import functools

import jax
import jax.numpy as jnp
from jax import lax
from jax.experimental import pallas as pl
from jax.experimental.pallas import tpu as pltpu
from jax.experimental.pallas import tpu_sc as plsc

N = 10000
E = 320000
D = 128
NC = 2
NS = 16
NW = NC * NS
CHUNK = 128
E_PAD = 327680
NCHUNK = E_PAD // CHUNK
NG0 = 125
NG1 = 35
assert NS * (NG0 + NG1) == NCHUNK
N_PAD = 10240
ZROWS = N_PAD // NS

_MESH = plsc.VectorSubcoreMesh(core_axis_name="c", subcore_axis_name="s")


def _fill_f32(ref, n, value):
    vec = jnp.full((16,), value, jnp.float32)
    for j in range(n // 16):
        ref[pl.ds(j * 16, 16)] = vec


def _copy_chunk_idx(src_ref, dst_ref, chunk):
    for g in range(CHUNK // 16):
        dst_ref[pl.ds(g * 16, 16)] = src_ref[pl.ds(chunk * CHUNK + g * 16, 16)]


def _deg_body(dst_hbm, zeros_hbm, degp_hbm, deg_sh, didx_all, didx_c, ones_v):
    c = lax.axis_index("c")
    s = lax.axis_index("s")
    _fill_f32(ones_v, CHUNK, 1.0)
    off = pl.multiple_of(s * ZROWS, 8)
    pltpu.sync_copy(zeros_hbm, deg_sh.at[pl.ds(off, ZROWS)])
    plsc.subcore_barrier()

    w = c * NS + s
    ept = NCHUNK // NW * CHUNK
    base = pl.multiple_of(w * ept, 8)
    pltpu.sync_copy(dst_hbm.at[pl.ds(base, ept)], didx_all)
    for i in range(NCHUNK // NW):
        _copy_chunk_idx(didx_all, didx_c, i)
        pltpu.sync_copy(ones_v, deg_sh.at[didx_c], add=True)

    plsc.subcore_barrier()
    pltpu.sync_copy(deg_sh.at[pl.ds(off, ZROWS)],
                    degp_hbm.at[c, pl.ds(off, ZROWS)])


def _scat_body(src_hbm, dst_hbm, xs_hbm, zeros_hbm, part_hbm,
               acc_sh, sidx_all, didx_a, didx_b,
               rows_a, rows_b, sem_a, sem_b, sem_da, sem_db):
    c = lax.axis_index("c")
    s = lax.axis_index("s")
    off = pl.multiple_of(s * ZROWS, 8)
    pltpu.sync_copy(zeros_hbm, acc_sh.at[pl.ds(off, ZROWS)])
    plsc.subcore_barrier()

    def run(ng, base):
        pltpu.sync_copy(src_hbm.at[pl.ds(base, ng * CHUNK)],
                        sidx_all.at[pl.ds(0, ng * CHUNK)])

        def gather_args(i):
            rows_v = rows_a if i % 2 == 0 else rows_b
            sem = sem_a if i % 2 == 0 else sem_b
            return (xs_hbm.at[sidx_all.at[pl.ds(i * CHUNK, CHUNK)]],
                    rows_v, sem)

        def didx_args(i):
            didx = didx_a if i % 2 == 0 else didx_b
            sem = sem_da if i % 2 == 0 else sem_db
            return dst_hbm.at[pl.ds(base + i * CHUNK, CHUNK)], didx, sem

        pltpu.async_copy(*gather_args(0))
        pltpu.async_copy(*didx_args(0))
        for i in range(ng):
            if i + 1 < ng:
                pltpu.async_copy(*gather_args(i + 1))
                pltpu.async_copy(*didx_args(i + 1))
            pltpu.make_async_copy(*gather_args(i)).wait()
            pltpu.make_async_copy(*didx_args(i)).wait()
            rows_v = rows_a if i % 2 == 0 else rows_b
            didx_v = didx_a if i % 2 == 0 else didx_b
            pltpu.sync_copy(rows_v, acc_sh.at[didx_v], add=True)

    @pl.when(c == 0)
    def _():
        run(NG0, pl.multiple_of(s * (NG0 * CHUNK), 8))

    @pl.when(c == 1)
    def _():
        run(NG1, pl.multiple_of(NS * NG0 * CHUNK + s * (NG1 * CHUNK), 8))

    plsc.subcore_barrier()
    pltpu.sync_copy(acc_sh.at[pl.ds(off, ZROWS)],
                    part_hbm.at[c, pl.ds(off, ZROWS)])


def _dis(d0, d1):
    deg = d0 + d1
    return jnp.where(deg > 0, lax.rsqrt(jnp.maximum(deg, 1e-12)), 0.0)


def _scale_body(x_ref, d0_ref, d1_ref, xs_ref):
    xs_ref[...] = x_ref[...] * _dis(d0_ref[...], d1_ref[...])


def _comb_body(p0_ref, p1_ref, d0_ref, d1_ref, out_ref):
    out_ref[...] = (p0_ref[...] + p1_ref[...]) * _dis(d0_ref[...], d1_ref[...])


_deg_kernel = pl.kernel(
    _deg_body,
    out_type=jax.ShapeDtypeStruct((NC, N_PAD), jnp.float32),
    mesh=_MESH,
    scratch_types=[
        pltpu.VMEM_SHARED((N_PAD,), jnp.float32),
        pltpu.VMEM((NCHUNK // NW * CHUNK,), jnp.int32),
        pltpu.VMEM((CHUNK,), jnp.int32),
        pltpu.VMEM((CHUNK,), jnp.float32),
    ],
)

_scat_kernel = pl.kernel(
    _scat_body,
    out_type=jax.ShapeDtypeStruct((NC, N_PAD, D), jnp.float32),
    mesh=_MESH,
    scratch_types=[
        pltpu.VMEM_SHARED((N_PAD, D), jnp.float32),
        pltpu.VMEM((NG0 * CHUNK,), jnp.int32),
        pltpu.VMEM((CHUNK,), jnp.int32),
        pltpu.VMEM((CHUNK,), jnp.int32),
        pltpu.VMEM((CHUNK, D), jnp.float32),
        pltpu.VMEM((CHUNK, D), jnp.float32),
        pltpu.SemaphoreType.DMA,
        pltpu.SemaphoreType.DMA,
        pltpu.SemaphoreType.DMA,
        pltpu.SemaphoreType.DMA,
    ],
)

_scale_call = pl.pallas_call(
    _scale_body, out_shape=jax.ShapeDtypeStruct((N, D), jnp.float32))

_comb_call = pl.pallas_call(
    _comb_body, out_shape=jax.ShapeDtypeStruct((N, D), jnp.float32))


def kernel(x, edge_index):
    ei = edge_index.astype(jnp.int32)
    npad = E_PAD - E
    src = jnp.concatenate([ei[0], jnp.zeros((npad,), jnp.int32)])
    dst = jnp.concatenate(
        [ei[1], N + jnp.arange(npad, dtype=jnp.int32) % (N_PAD - N)])
    zeros = jnp.zeros((ZROWS, D), jnp.float32)

    degp = _deg_kernel(dst, zeros[:, 0])
    d0 = degp[0, :N][:, None]
    d1 = degp[1, :N][:, None]

    xs = _scale_call(x, d0, d1)
    part = _scat_kernel(src, dst, xs, zeros)
    return _comb_call(part[0, :N], part[1, :N], d0, d1)
